# Initial kernel scaffold; baseline (speedup 1.0000x reference)
import jax
import jax.numpy as jnp
from jax.experimental import pallas as pl


def kernel(ent_emb, rel_emb, edge_index, edge_rel, edge_inv, W_I, b_I, W_O, b_O, W_S, b_S, W_R, b_R):
    raise NotImplementedError("write your pallas kernel here")


# speedup vs baseline: 2.8709x; 2.8709x over previous
import functools

import jax
import jax.numpy as jnp
from jax import lax
from jax.experimental import pallas as pl
from jax.experimental.pallas import tpu as pltpu
from jax.experimental.pallas import tpu_sc as plsc

N = 10000
NPAD = 10240
ENT = 128
REL = 128
RPAD = 512
NC = 2
NS = 16
L = 16
CHUNK = 80
NHALF = NPAD // NC
NACC = 5440
NZP = NACC // CHUNK
FSTRIPE = NHALF // NS
BA = 2048
DH = 80
DL = 128
EB = 3200


def _prep_ent_body(x_ref, wi_ref, wo_ref, ws_ref, bs_ref, gh_ref, s_ref):
    x = x_ref[...]
    gh_ref[0] = jnp.dot(x, wi_ref[...], preferred_element_type=jnp.float32)
    gh_ref[1] = jnp.dot(x, wo_ref[...], preferred_element_type=jnp.float32)
    s_ref[...] = jnp.dot(x, ws_ref[...], preferred_element_type=jnp.float32) + bs_ref[...]


def _prep_rel_body(r_ref, wi_ref, wo_ref, wr_ref, bi_ref, bo_ref, br_ref,
                   pq_ref, rn_ref):
    r = r_ref[...]
    pq_ref[0] = jnp.dot(r, wi_ref[...], preferred_element_type=jnp.float32) + bi_ref[...]
    pq_ref[1] = jnp.dot(r, wo_ref[...], preferred_element_type=jnp.float32) + bo_ref[...]
    rn_ref[...] = jnp.dot(r, wr_ref[...], preferred_element_type=jnp.float32) + br_ref[...]


def _deg_hist_body(dst_ref, out_ref):
    d = dst_ref[0]
    hi = d // DL
    lo = d - hi * DL
    oh_hi = (hi ==
             lax.broadcasted_iota(jnp.int32, (EB, DH), 1)).astype(jnp.float32)
    oh_lo = (lo ==
             lax.broadcasted_iota(jnp.int32, (EB, DL), 1)).astype(jnp.float32)
    blk = lax.dot_general(oh_hi, oh_lo, (((0,), (0,)), ((), ())),
                          preferred_element_type=jnp.float32)

    @pl.when(pl.program_id(0) == 0)
    def _():
        out_ref[...] = jnp.zeros_like(out_ref)

    out_ref[...] += blk


def _post_body(acc_ref, deg_ref, s_ref, out_ref):
    deg = deg_ref[...]
    out_ref[...] = s_ref[...] + acc_ref[...] / jnp.maximum(deg, 1.0)


def _edge_sc_kernel(e_total):
    ew = e_total // NS
    n_chunks = ew // CHUNK
    mesh = plsc.VectorSubcoreMesh(core_axis_name="c", subcore_axis_name="s")

    @functools.partial(
        pl.kernel,
        mesh=mesh,
        out_type=jax.ShapeDtypeStruct((NPAD, ENT), jnp.float32),
        scratch_types=[
            pltpu.VMEM_SHARED((NACC, ENT), jnp.float32),
            pltpu.VMEM((CHUNK,), jnp.int32),
            pltpu.VMEM((CHUNK,), jnp.int32),
            pltpu.VMEM((CHUNK,), jnp.int32),
            pltpu.VMEM((CHUNK, ENT), jnp.float32),
            pltpu.VMEM((CHUNK, ENT), jnp.float32),
            pltpu.SemaphoreType.DMA,
            pltpu.SemaphoreType.DMA,
        ],
    )
    def sck(pq_hbm, gh_hbm, ipq_hbm, igh_hbm, dst_hbm, zrow_hbm,
            acc_out,
            acc_sh, ipq_v, igh_v, dst_v, bufpq, bufgh, sem1, sem2):
        cid = lax.axis_index("c")
        sid = lax.axis_index("s")
        dst_lo = cid * NHALF

        pltpu.sync_copy(zrow_hbm, bufpq)
        for k in range(-(-NZP // NS)):
            p_static = k * NS
            @pl.when(p_static + sid < NZP)
            def _():
                pltpu.sync_copy(bufpq,
                                acc_sh.at[pl.ds((p_static + sid) * CHUNK, CHUNK)])
        plsc.subcore_barrier()

        def body(c, carry):
            off = sid * ew + c * CHUNK
            pltpu.sync_copy(ipq_hbm.at[pl.ds(off, CHUNK)], ipq_v)
            pltpu.sync_copy(igh_hbm.at[pl.ds(off, CHUNK)], igh_v)
            pltpu.sync_copy(dst_hbm.at[pl.ds(off, CHUNK)], dst_v)
            pltpu.async_copy(pq_hbm.at[ipq_v], bufpq, sem1).wait()
            pltpu.async_copy(gh_hbm.at[igh_v], bufgh, sem2).wait()
            for j in range(CHUNK // L):
                d = dst_v[pl.ds(j * L, L)] - dst_lo
                oob = (d < 0) | (d >= NHALF)
                dst_v[pl.ds(j * L, L)] = jnp.where(oob, NHALF, d)
            pltpu.sync_copy(bufpq, acc_sh.at[dst_v], add=True)
            pltpu.sync_copy(bufgh, acc_sh.at[dst_v], add=True)
            return carry

        lax.fori_loop(0, n_chunks, body, 0)
        plsc.subcore_barrier()

        row0 = sid * FSTRIPE
        out0 = cid * NHALF + row0
        for k in range(FSTRIPE // CHUNK):
            pltpu.sync_copy(acc_sh.at[pl.ds(row0 + k * CHUNK, CHUNK)], bufpq)
            pltpu.sync_copy(bufpq, acc_out.at[pl.ds(out0 + k * CHUNK, CHUNK)])

    return sck


def kernel(ent_emb, rel_emb, edge_index, edge_rel, edge_inv,
           W_I, b_I, W_O, b_O, W_S, b_S, W_R, b_R):
    n, ent_d = ent_emb.shape
    r_cnt, rel_d = rel_emb.shape
    e_total = edge_rel.shape[0]
    assert (n, ent_d, rel_d) == (N, ENT, REL)
    assert e_total % (NS * CHUNK) == 0 and e_total % EB == 0

    src = edge_index[0]
    dst = edge_index[1]

    ent_pad = jnp.zeros((NPAD, ENT), jnp.float32).at[:n].set(ent_emb)
    gh, s_out = pl.pallas_call(
        _prep_ent_body,
        grid=(NPAD // BA,),
        in_specs=[
            pl.BlockSpec((BA, ENT), lambda i: (i, 0)),
            pl.BlockSpec((ENT, ENT), lambda i: (0, 0)),
            pl.BlockSpec((ENT, ENT), lambda i: (0, 0)),
            pl.BlockSpec((ENT, ENT), lambda i: (0, 0)),
            pl.BlockSpec((1, ENT), lambda i: (0, 0)),
        ],
        out_specs=[
            pl.BlockSpec((2, BA, ENT), lambda i: (0, i, 0)),
            pl.BlockSpec((BA, ENT), lambda i: (i, 0)),
        ],
        out_shape=[
            jax.ShapeDtypeStruct((2, NPAD, ENT), jnp.float32),
            jax.ShapeDtypeStruct((NPAD, ENT), jnp.float32),
        ],
    )(ent_pad, W_I[:, REL:].T, W_O[:, REL:].T, W_S.T, b_S[None, :])

    rel_pad = jnp.zeros((RPAD, rel_d), jnp.float32).at[:r_cnt].set(rel_emb)
    pq, rel_new_pad = pl.pallas_call(
        _prep_rel_body,
        out_shape=[
            jax.ShapeDtypeStruct((2, RPAD, ENT), jnp.float32),
            jax.ShapeDtypeStruct((RPAD, REL), jnp.float32),
        ],
    )(rel_pad, W_I[:, :REL].T, W_O[:, :REL].T, W_R.T,
      b_I[None, :], b_O[None, :], b_R[None, :])

    deg2d = pl.pallas_call(
        _deg_hist_body,
        grid=(e_total // EB,),
        in_specs=[pl.BlockSpec((1, EB, 1), lambda k: (k, 0, 0))],
        out_specs=pl.BlockSpec((DH, DL), lambda k: (0, 0)),
        out_shape=jax.ShapeDtypeStruct((DH, DL), jnp.float32),
    )(dst.reshape(e_total // EB, EB, 1))

    idx_pq = edge_rel + edge_inv * RPAD
    idx_gh = src + edge_inv * NPAD
    zrow = jnp.zeros((CHUNK, ENT), jnp.float32)
    acc = _edge_sc_kernel(e_total)(
        pq.reshape(2 * RPAD, ENT), gh.reshape(2 * NPAD, ENT),
        idx_pq, idx_gh, dst, zrow)

    ent_new_pad = pl.pallas_call(
        _post_body,
        grid=(NPAD // BA,),
        in_specs=[
            pl.BlockSpec((BA, ENT), lambda i: (i, 0)),
            pl.BlockSpec((BA, 1), lambda i: (i, 0)),
            pl.BlockSpec((BA, ENT), lambda i: (i, 0)),
        ],
        out_specs=pl.BlockSpec((BA, ENT), lambda i: (i, 0)),
        out_shape=jax.ShapeDtypeStruct((NPAD, ENT), jnp.float32),
    )(acc, deg2d.reshape(NPAD, 1), s_out)

    return ent_new_pad[:n], rel_new_pad[:r_cnt]


# speedup vs baseline: 4.7683x; 1.6609x over previous
import functools

import jax
import jax.numpy as jnp
from jax import lax
from jax.experimental import pallas as pl
from jax.experimental.pallas import tpu as pltpu
from jax.experimental.pallas import tpu_sc as plsc

N = 10000
NPAD = 10240
ENT = 128
REL = 128
RPAD = 512
NC = 2
NS = 16
L = 16
CHUNK = 80
NHALF = NPAD // NC
NACC = 5200
NZP = NACC // CHUNK
FSTRIPE = NHALF // NS
BA = 2048
DH = 80
DL = 128
EB = 3200


def _prep_ent_body(x_ref, wi_ref, wo_ref, ws_ref, bs_ref, gh_ref, s_ref):
    x = x_ref[...]
    gh_ref[0] = jnp.dot(x, wi_ref[...], preferred_element_type=jnp.float32)
    gh_ref[1] = jnp.dot(x, wo_ref[...], preferred_element_type=jnp.float32)
    s_ref[...] = jnp.dot(x, ws_ref[...], preferred_element_type=jnp.float32) + bs_ref[...]


def _prep_rel_body(r_ref, wi_ref, wo_ref, wr_ref, bi_ref, bo_ref, br_ref,
                   pq_ref, rn_ref):
    r = r_ref[...]
    pq_ref[0] = jnp.dot(r, wi_ref[...], preferred_element_type=jnp.float32) + bi_ref[...]
    pq_ref[1] = jnp.dot(r, wo_ref[...], preferred_element_type=jnp.float32) + bo_ref[...]
    rn_ref[...] = jnp.dot(r, wr_ref[...], preferred_element_type=jnp.float32) + br_ref[...]


def _deg_hist_body(dst_ref, out_ref):
    d = dst_ref[0]
    hi = d // DL
    lo = d - hi * DL
    oh_hi = (hi ==
             lax.broadcasted_iota(jnp.int32, (EB, DH), 1)).astype(jnp.float32)
    oh_lo = (lo ==
             lax.broadcasted_iota(jnp.int32, (EB, DL), 1)).astype(jnp.float32)
    blk = lax.dot_general(oh_hi, oh_lo, (((0,), (0,)), ((), ())),
                          preferred_element_type=jnp.float32)

    @pl.when(pl.program_id(0) == 0)
    def _():
        out_ref[...] = jnp.zeros_like(out_ref)

    out_ref[...] += blk


def _post_body(acc_ref, deg_ref, s_ref, out_ref):
    deg = deg_ref[...]
    out_ref[...] = s_ref[...] + acc_ref[...] / jnp.maximum(deg, 1.0)


def _edge_sc_kernel(e_total):
    ew = e_total // NS
    n_chunks = ew // CHUNK
    mesh = plsc.VectorSubcoreMesh(core_axis_name="c", subcore_axis_name="s")

    assert n_chunks % 2 == 0 and n_chunks >= 4

    @functools.partial(
        pl.kernel,
        mesh=mesh,
        out_type=jax.ShapeDtypeStruct((NPAD, ENT), jnp.float32),
        scratch_types=[
            pltpu.VMEM_SHARED((NACC, ENT), jnp.float32),
            pltpu.VMEM((CHUNK,), jnp.int32),
            pltpu.VMEM((CHUNK,), jnp.int32),
            pltpu.VMEM((CHUNK,), jnp.int32),
            pltpu.VMEM((CHUNK,), jnp.int32),
            pltpu.VMEM((CHUNK,), jnp.int32),
            pltpu.VMEM((CHUNK,), jnp.int32),
            pltpu.VMEM((CHUNK, ENT), jnp.float32),
            pltpu.VMEM((CHUNK, ENT), jnp.float32),
            pltpu.VMEM((CHUNK, ENT), jnp.float32),
            pltpu.VMEM((CHUNK, ENT), jnp.float32),
            pltpu.SemaphoreType.DMA,
            pltpu.SemaphoreType.DMA,
            pltpu.SemaphoreType.DMA,
            pltpu.SemaphoreType.DMA,
        ],
    )
    def sck(pq_hbm, gh_hbm, ipq_hbm, igh_hbm, dst_hbm, zrow_hbm,
            acc_out,
            acc_sh, ipqA, ighA, dstA, ipqB, ighB, dstB,
            bufpqA, bufghA, bufpqB, bufghB, semA1, semA2, semB1, semB2):
        cid = lax.axis_index("c")
        sid = lax.axis_index("s")
        dst_lo = cid * NHALF

        pltpu.sync_copy(zrow_hbm, bufpqA)
        for k in range(-(-NZP // NS)):
            p_static = k * NS
            @pl.when(p_static + sid < NZP)
            def _():
                pltpu.sync_copy(bufpqA,
                                acc_sh.at[pl.ds((p_static + sid) * CHUNK, CHUNK)])
        plsc.subcore_barrier()

        def prefetch(c, ipq_v, igh_v, dst_v, bufpq, bufgh, sem1, sem2):
            off = sid * ew + c * CHUNK
            pltpu.sync_copy(ipq_hbm.at[pl.ds(off, CHUNK)], ipq_v)
            pltpu.sync_copy(igh_hbm.at[pl.ds(off, CHUNK)], igh_v)
            pltpu.sync_copy(dst_hbm.at[pl.ds(off, CHUNK)], dst_v)
            cp1 = pltpu.async_copy(pq_hbm.at[ipq_v], bufpq, sem1)
            cp2 = pltpu.async_copy(gh_hbm.at[igh_v], bufgh, sem2)
            for j in range(CHUNK // L):
                d = dst_v[pl.ds(j * L, L)] - dst_lo
                oob = (d < 0) | (d >= NHALF)
                dst_v[pl.ds(j * L, L)] = jnp.where(oob, NHALF, d)
            return cp1, cp2

        def commit(dst_v, bufpq, bufgh, cp1, cp2):
            cp1.wait()
            cp2.wait()
            pltpu.sync_copy(bufpq, acc_sh.at[dst_v], add=True)
            pltpu.sync_copy(bufgh, acc_sh.at[dst_v], add=True)

        cpA = prefetch(0, ipqA, ighA, dstA, bufpqA, bufghA, semA1, semA2)

        def body(i, carry):
            cB = 2 * i + 1
            cpB = prefetch(cB, ipqB, ighB, dstB, bufpqB, bufghB, semB1, semB2)
            commit(dstA, bufpqA, bufghA,
                   pltpu.make_async_copy(pq_hbm.at[ipqA], bufpqA, semA1),
                   pltpu.make_async_copy(gh_hbm.at[ighA], bufghA, semA2))
            cpA2 = prefetch(cB + 1, ipqA, ighA, dstA, bufpqA, bufghA,
                            semA1, semA2)
            commit(dstB, bufpqB, bufghB,
                   pltpu.make_async_copy(pq_hbm.at[ipqB], bufpqB, semB1),
                   pltpu.make_async_copy(gh_hbm.at[ighB], bufghB, semB2))
            return carry

        lax.fori_loop(0, n_chunks // 2 - 1, body, 0)
        cpB = prefetch(n_chunks - 1, ipqB, ighB, dstB, bufpqB, bufghB,
                       semB1, semB2)
        commit(dstA, bufpqA, bufghA,
               pltpu.make_async_copy(pq_hbm.at[ipqA], bufpqA, semA1),
               pltpu.make_async_copy(gh_hbm.at[ighA], bufghA, semA2))
        commit(dstB, bufpqB, bufghB, cpB[0], cpB[1])
        plsc.subcore_barrier()

        row0 = sid * FSTRIPE
        out0 = cid * NHALF + row0
        for k in range(FSTRIPE // CHUNK):
            pltpu.sync_copy(acc_sh.at[pl.ds(row0 + k * CHUNK, CHUNK)], bufpqA)
            pltpu.sync_copy(bufpqA, acc_out.at[pl.ds(out0 + k * CHUNK, CHUNK)])

    return sck


def kernel(ent_emb, rel_emb, edge_index, edge_rel, edge_inv,
           W_I, b_I, W_O, b_O, W_S, b_S, W_R, b_R):
    n, ent_d = ent_emb.shape
    r_cnt, rel_d = rel_emb.shape
    e_total = edge_rel.shape[0]
    assert (n, ent_d, rel_d) == (N, ENT, REL)
    assert e_total % (NS * CHUNK) == 0 and e_total % EB == 0

    src = edge_index[0]
    dst = edge_index[1]

    ent_pad = jnp.zeros((NPAD, ENT), jnp.float32).at[:n].set(ent_emb)
    gh, s_out = pl.pallas_call(
        _prep_ent_body,
        grid=(NPAD // BA,),
        in_specs=[
            pl.BlockSpec((BA, ENT), lambda i: (i, 0)),
            pl.BlockSpec((ENT, ENT), lambda i: (0, 0)),
            pl.BlockSpec((ENT, ENT), lambda i: (0, 0)),
            pl.BlockSpec((ENT, ENT), lambda i: (0, 0)),
            pl.BlockSpec((1, ENT), lambda i: (0, 0)),
        ],
        out_specs=[
            pl.BlockSpec((2, BA, ENT), lambda i: (0, i, 0)),
            pl.BlockSpec((BA, ENT), lambda i: (i, 0)),
        ],
        out_shape=[
            jax.ShapeDtypeStruct((2, NPAD, ENT), jnp.float32),
            jax.ShapeDtypeStruct((NPAD, ENT), jnp.float32),
        ],
    )(ent_pad, W_I[:, REL:].T, W_O[:, REL:].T, W_S.T, b_S[None, :])

    rel_pad = jnp.zeros((RPAD, rel_d), jnp.float32).at[:r_cnt].set(rel_emb)
    pq, rel_new_pad = pl.pallas_call(
        _prep_rel_body,
        out_shape=[
            jax.ShapeDtypeStruct((2, RPAD, ENT), jnp.float32),
            jax.ShapeDtypeStruct((RPAD, REL), jnp.float32),
        ],
    )(rel_pad, W_I[:, :REL].T, W_O[:, :REL].T, W_R.T,
      b_I[None, :], b_O[None, :], b_R[None, :])

    deg2d = pl.pallas_call(
        _deg_hist_body,
        grid=(e_total // EB,),
        in_specs=[pl.BlockSpec((1, EB, 1), lambda k: (k, 0, 0))],
        out_specs=pl.BlockSpec((DH, DL), lambda k: (0, 0)),
        out_shape=jax.ShapeDtypeStruct((DH, DL), jnp.float32),
    )(dst.reshape(e_total // EB, EB, 1))

    idx_pq = edge_rel + edge_inv * RPAD
    idx_gh = src + edge_inv * NPAD
    zrow = jnp.zeros((CHUNK, ENT), jnp.float32)
    acc = _edge_sc_kernel(e_total)(
        pq.reshape(2 * RPAD, ENT), gh.reshape(2 * NPAD, ENT),
        idx_pq, idx_gh, dst, zrow)

    ent_new_pad = pl.pallas_call(
        _post_body,
        grid=(NPAD // BA,),
        in_specs=[
            pl.BlockSpec((BA, ENT), lambda i: (i, 0)),
            pl.BlockSpec((BA, 1), lambda i: (i, 0)),
            pl.BlockSpec((BA, ENT), lambda i: (i, 0)),
        ],
        out_specs=pl.BlockSpec((BA, ENT), lambda i: (i, 0)),
        out_shape=jax.ShapeDtypeStruct((NPAD, ENT), jnp.float32),
    )(acc, deg2d.reshape(NPAD, 1), s_out)

    return ent_new_pad[:n], rel_new_pad[:r_cnt]


# speedup vs baseline: 5.7500x; 1.2059x over previous
import functools

import jax
import jax.numpy as jnp
from jax import lax
from jax.experimental import pallas as pl
from jax.experimental.pallas import tpu as pltpu
from jax.experimental.pallas import tpu_sc as plsc

N = 10000
NPAD = 10240
ENT = 128
REL = 128
RPAD = 512
NC = 2
NS = 16
L = 16
CHUNK = 80
NHALF = NPAD // NC
NACC = 5200
NZP = NACC // CHUNK
FSTRIPE = NHALF // NS
BA = 2048
DH = 80
DL = 128
EB = 3200


def _prep_ent_body(x_ref, wi_ref, wo_ref, ws_ref, bs_ref, gh_ref, s_ref):
    x = x_ref[...]
    gh_ref[0] = jnp.dot(x, wi_ref[...], preferred_element_type=jnp.float32)
    gh_ref[1] = jnp.dot(x, wo_ref[...], preferred_element_type=jnp.float32)
    s_ref[...] = jnp.dot(x, ws_ref[...], preferred_element_type=jnp.float32) + bs_ref[...]


def _prep_rel_body(r_ref, wi_ref, wo_ref, wr_ref, bi_ref, bo_ref, br_ref,
                   pq_ref, rn_ref):
    r = r_ref[...]
    pq_ref[0] = jnp.dot(r, wi_ref[...], preferred_element_type=jnp.float32) + bi_ref[...]
    pq_ref[1] = jnp.dot(r, wo_ref[...], preferred_element_type=jnp.float32) + bo_ref[...]
    rn_ref[...] = jnp.dot(r, wr_ref[...], preferred_element_type=jnp.float32) + br_ref[...]


def _deg_hist_body(dst_ref, out_ref):
    d = dst_ref[0]
    hi = d // DL
    lo = d - hi * DL
    oh_hi = (hi ==
             lax.broadcasted_iota(jnp.int32, (EB, DH), 1)).astype(jnp.float32)
    oh_lo = (lo ==
             lax.broadcasted_iota(jnp.int32, (EB, DL), 1)).astype(jnp.float32)
    blk = lax.dot_general(oh_hi, oh_lo, (((0,), (0,)), ((), ())),
                          preferred_element_type=jnp.float32)

    @pl.when(pl.program_id(0) == 0)
    def _():
        out_ref[...] = jnp.zeros_like(out_ref)

    out_ref[...] += blk


def _post_body(acc_ref, deg_ref, s_ref, out_ref):
    deg = deg_ref[...]
    out_ref[...] = s_ref[...] + acc_ref[...] / jnp.maximum(deg, 1.0)


def _edge_sc_kernel(e_total):
    ew = e_total // NS
    n_chunks = ew // CHUNK
    mesh = plsc.VectorSubcoreMesh(core_axis_name="c", subcore_axis_name="s")

    assert n_chunks % 2 == 0 and n_chunks >= 4

    @functools.partial(
        pl.kernel,
        mesh=mesh,
        out_type=jax.ShapeDtypeStruct((NPAD, ENT), jnp.float32),
        scratch_types=[
            pltpu.VMEM_SHARED((NACC, ENT), jnp.float32),
            pltpu.VMEM((CHUNK,), jnp.int32),
            pltpu.VMEM((CHUNK,), jnp.int32),
            pltpu.VMEM((CHUNK,), jnp.int32),
            pltpu.VMEM((CHUNK,), jnp.int32),
            pltpu.VMEM((CHUNK,), jnp.int32),
            pltpu.VMEM((CHUNK,), jnp.int32),
            pltpu.VMEM((CHUNK, ENT), jnp.float32),
            pltpu.VMEM((CHUNK, ENT), jnp.float32),
            pltpu.VMEM((CHUNK, ENT), jnp.float32),
            pltpu.VMEM((CHUNK, ENT), jnp.float32),
            pltpu.SemaphoreType.DMA,
            pltpu.SemaphoreType.DMA,
            pltpu.SemaphoreType.DMA,
            pltpu.SemaphoreType.DMA,
            pltpu.SemaphoreType.DMA,
            pltpu.SemaphoreType.DMA,
        ],
    )
    def sck(pq_hbm, gh_hbm, ipq_hbm, igh_hbm, dst_hbm, zrow_hbm,
            acc_out,
            acc_sh, ipqA, ighA, dstA, ipqB, ighB, dstB,
            bufpqA, bufghA, bufpqB, bufghB,
            semA1, semA2, semB1, semB2, semA0, semB0):
        cid = lax.axis_index("c")
        sid = lax.axis_index("s")
        dst_lo = cid * NHALF

        pltpu.sync_copy(zrow_hbm, bufpqA)
        for k in range(-(-NZP // NS)):
            p_static = k * NS
            @pl.when(p_static + sid < NZP)
            def _():
                pltpu.sync_copy(bufpqA,
                                acc_sh.at[pl.ds((p_static + sid) * CHUNK, CHUNK)])
        plsc.subcore_barrier()

        def prefetch(c, ipq_v, igh_v, dst_v, bufpq, bufgh, sem0, sem1, sem2):
            off = sid * ew + c * CHUNK
            ci1 = pltpu.async_copy(ipq_hbm.at[pl.ds(off, CHUNK)], ipq_v, sem0)
            ci2 = pltpu.async_copy(igh_hbm.at[pl.ds(off, CHUNK)], igh_v, sem0)
            ci3 = pltpu.async_copy(dst_hbm.at[pl.ds(off, CHUNK)], dst_v, sem0)
            ci1.wait()
            ci2.wait()
            ci3.wait()
            cp1 = pltpu.async_copy(pq_hbm.at[ipq_v], bufpq, sem1)
            cp2 = pltpu.async_copy(gh_hbm.at[igh_v], bufgh, sem2)
            for j in range(CHUNK // L):
                d = dst_v[pl.ds(j * L, L)] - dst_lo
                oob = (d < 0) | (d >= NHALF)
                dst_v[pl.ds(j * L, L)] = jnp.where(oob, NHALF, d)
            return cp1, cp2

        def commit(dst_v, bufpq, bufgh, cp1, cp2):
            cp1.wait()
            cp2.wait()
            pltpu.sync_copy(bufpq, acc_sh.at[dst_v], add=True)
            pltpu.sync_copy(bufgh, acc_sh.at[dst_v], add=True)

        cpA = prefetch(0, ipqA, ighA, dstA, bufpqA, bufghA, semA0, semA1, semA2)

        def body(i, carry):
            cB = 2 * i + 1
            cpB = prefetch(cB, ipqB, ighB, dstB, bufpqB, bufghB, semB0, semB1, semB2)
            commit(dstA, bufpqA, bufghA,
                   pltpu.make_async_copy(pq_hbm.at[ipqA], bufpqA, semA1),
                   pltpu.make_async_copy(gh_hbm.at[ighA], bufghA, semA2))
            cpA2 = prefetch(cB + 1, ipqA, ighA, dstA, bufpqA, bufghA,
                            semA0, semA1, semA2)
            commit(dstB, bufpqB, bufghB,
                   pltpu.make_async_copy(pq_hbm.at[ipqB], bufpqB, semB1),
                   pltpu.make_async_copy(gh_hbm.at[ighB], bufghB, semB2))
            return carry

        lax.fori_loop(0, n_chunks // 2 - 1, body, 0)
        cpB = prefetch(n_chunks - 1, ipqB, ighB, dstB, bufpqB, bufghB,
                       semB0, semB1, semB2)
        commit(dstA, bufpqA, bufghA,
               pltpu.make_async_copy(pq_hbm.at[ipqA], bufpqA, semA1),
               pltpu.make_async_copy(gh_hbm.at[ighA], bufghA, semA2))
        commit(dstB, bufpqB, bufghB, cpB[0], cpB[1])
        plsc.subcore_barrier()

        row0 = sid * FSTRIPE
        out0 = cid * NHALF + row0
        for k in range(FSTRIPE // CHUNK):
            pltpu.sync_copy(acc_sh.at[pl.ds(row0 + k * CHUNK, CHUNK)], bufpqA)
            pltpu.sync_copy(bufpqA, acc_out.at[pl.ds(out0 + k * CHUNK, CHUNK)])

    return sck


def kernel(ent_emb, rel_emb, edge_index, edge_rel, edge_inv,
           W_I, b_I, W_O, b_O, W_S, b_S, W_R, b_R):
    n, ent_d = ent_emb.shape
    r_cnt, rel_d = rel_emb.shape
    e_total = edge_rel.shape[0]
    assert (n, ent_d, rel_d) == (N, ENT, REL)
    assert e_total % (NS * CHUNK) == 0 and e_total % EB == 0

    src = edge_index[0]
    dst = edge_index[1]

    ent_pad = jnp.zeros((NPAD, ENT), jnp.float32).at[:n].set(ent_emb)
    gh, s_out = pl.pallas_call(
        _prep_ent_body,
        grid=(NPAD // BA,),
        in_specs=[
            pl.BlockSpec((BA, ENT), lambda i: (i, 0)),
            pl.BlockSpec((ENT, ENT), lambda i: (0, 0)),
            pl.BlockSpec((ENT, ENT), lambda i: (0, 0)),
            pl.BlockSpec((ENT, ENT), lambda i: (0, 0)),
            pl.BlockSpec((1, ENT), lambda i: (0, 0)),
        ],
        out_specs=[
            pl.BlockSpec((2, BA, ENT), lambda i: (0, i, 0)),
            pl.BlockSpec((BA, ENT), lambda i: (i, 0)),
        ],
        out_shape=[
            jax.ShapeDtypeStruct((2, NPAD, ENT), jnp.float32),
            jax.ShapeDtypeStruct((NPAD, ENT), jnp.float32),
        ],
    )(ent_pad, W_I[:, REL:].T, W_O[:, REL:].T, W_S.T, b_S[None, :])

    rel_pad = jnp.zeros((RPAD, rel_d), jnp.float32).at[:r_cnt].set(rel_emb)
    pq, rel_new_pad = pl.pallas_call(
        _prep_rel_body,
        out_shape=[
            jax.ShapeDtypeStruct((2, RPAD, ENT), jnp.float32),
            jax.ShapeDtypeStruct((RPAD, REL), jnp.float32),
        ],
    )(rel_pad, W_I[:, :REL].T, W_O[:, :REL].T, W_R.T,
      b_I[None, :], b_O[None, :], b_R[None, :])

    deg2d = pl.pallas_call(
        _deg_hist_body,
        grid=(e_total // EB,),
        in_specs=[pl.BlockSpec((1, EB, 1), lambda k: (k, 0, 0))],
        out_specs=pl.BlockSpec((DH, DL), lambda k: (0, 0)),
        out_shape=jax.ShapeDtypeStruct((DH, DL), jnp.float32),
    )(dst.reshape(e_total // EB, EB, 1))

    idx_pq = edge_rel + edge_inv * RPAD
    idx_gh = src + edge_inv * NPAD
    zrow = jnp.zeros((CHUNK, ENT), jnp.float32)
    acc = _edge_sc_kernel(e_total)(
        pq.reshape(2 * RPAD, ENT), gh.reshape(2 * NPAD, ENT),
        idx_pq, idx_gh, dst, zrow)

    ent_new_pad = pl.pallas_call(
        _post_body,
        grid=(NPAD // BA,),
        in_specs=[
            pl.BlockSpec((BA, ENT), lambda i: (i, 0)),
            pl.BlockSpec((BA, 1), lambda i: (i, 0)),
            pl.BlockSpec((BA, ENT), lambda i: (i, 0)),
        ],
        out_specs=pl.BlockSpec((BA, ENT), lambda i: (i, 0)),
        out_shape=jax.ShapeDtypeStruct((NPAD, ENT), jnp.float32),
    )(acc, deg2d.reshape(NPAD, 1), s_out)

    return ent_new_pad[:n], rel_new_pad[:r_cnt]


# speedup vs baseline: 5.7740x; 1.0042x over previous
import functools

import jax
import jax.numpy as jnp
from jax import lax
from jax.experimental import pallas as pl
from jax.experimental.pallas import tpu as pltpu
from jax.experimental.pallas import tpu_sc as plsc

N = 10000
NPAD = 10240
ENT = 128
REL = 128
RPAD = 512
NC = 2
NS = 16
L = 16
CHUNK = 80
NHALF = NPAD // NC
NACC = 5200
NZP = NACC // CHUNK
FSTRIPE = NHALF // NS
BA = 2048
DH = 80
DL = 128
EB = 3200


def _prep_ent_body(x_ref, wi_ref, wo_ref, ws_ref, bs_ref, gh_ref, s_ref):
    x = x_ref[...]
    gh_ref[0] = jnp.dot(x, wi_ref[...], preferred_element_type=jnp.float32)
    gh_ref[1] = jnp.dot(x, wo_ref[...], preferred_element_type=jnp.float32)
    s_ref[...] = jnp.dot(x, ws_ref[...], preferred_element_type=jnp.float32) + bs_ref[...]


def _prep_rel_body(r_ref, wi_ref, wo_ref, wr_ref, bi_ref, bo_ref, br_ref,
                   pq_ref, rn_ref):
    r = r_ref[...]
    pq_ref[0] = jnp.dot(r, wi_ref[...], preferred_element_type=jnp.float32) + bi_ref[...]
    pq_ref[1] = jnp.dot(r, wo_ref[...], preferred_element_type=jnp.float32) + bo_ref[...]
    rn_ref[...] = jnp.dot(r, wr_ref[...], preferred_element_type=jnp.float32) + br_ref[...]


def _deg_hist_body(dst_ref, out_ref):
    d = dst_ref[0]
    hi = d // DL
    lo = d - hi * DL
    oh_hi = (hi ==
             lax.broadcasted_iota(jnp.int32, (EB, DH), 1)).astype(jnp.float32)
    oh_lo = (lo ==
             lax.broadcasted_iota(jnp.int32, (EB, DL), 1)).astype(jnp.float32)
    blk = lax.dot_general(oh_hi, oh_lo, (((0,), (0,)), ((), ())),
                          preferred_element_type=jnp.float32)

    @pl.when(pl.program_id(0) == 0)
    def _():
        out_ref[...] = jnp.zeros_like(out_ref)

    out_ref[...] += blk


def _post_body(acc_ref, deg_ref, s_ref, out_ref):
    deg = deg_ref[...]
    out_ref[...] = s_ref[...] + acc_ref[...] / jnp.maximum(deg, 1.0)


def _edge_sc_kernel(e_total):
    ew = e_total // NS
    n_chunks = ew // CHUNK
    mesh = plsc.VectorSubcoreMesh(core_axis_name="c", subcore_axis_name="s")

    assert n_chunks % 2 == 0 and n_chunks >= 4

    @functools.partial(
        pl.kernel,
        mesh=mesh,
        out_type=jax.ShapeDtypeStruct((NPAD, ENT), jnp.float32),
        scratch_types=[
            pltpu.VMEM_SHARED((NACC, ENT), jnp.float32),
            pltpu.VMEM((CHUNK,), jnp.int32),
            pltpu.VMEM((CHUNK,), jnp.int32),
            pltpu.VMEM((CHUNK,), jnp.int32),
            pltpu.VMEM((CHUNK,), jnp.int32),
            pltpu.VMEM((CHUNK,), jnp.int32),
            pltpu.VMEM((CHUNK,), jnp.int32),
            pltpu.VMEM((CHUNK, ENT), jnp.float32),
            pltpu.VMEM((CHUNK, ENT), jnp.float32),
            pltpu.VMEM((CHUNK, ENT), jnp.float32),
            pltpu.VMEM((CHUNK, ENT), jnp.float32),
            pltpu.SemaphoreType.DMA,
            pltpu.SemaphoreType.DMA,
            pltpu.SemaphoreType.DMA,
            pltpu.SemaphoreType.DMA,
            pltpu.SemaphoreType.DMA,
            pltpu.SemaphoreType.DMA,
        ],
    )
    def sck(pq_hbm, gh_hbm, ipq_hbm, igh_hbm, dst_hbm, zrow_hbm,
            acc_out,
            acc_sh, ipqA, ighA, dstA, ipqB, ighB, dstB,
            bufpqA, bufghA, bufpqB, bufghB,
            semA1, semA2, semB1, semB2, semA0, semB0):
        cid = lax.axis_index("c")
        sid = lax.axis_index("s")
        dst_lo = cid * NHALF

        pltpu.sync_copy(zrow_hbm, bufpqA)
        for k in range(-(-NZP // NS)):
            p_static = k * NS
            @pl.when(p_static + sid < NZP)
            def _():
                pltpu.sync_copy(bufpqA,
                                acc_sh.at[pl.ds((p_static + sid) * CHUNK, CHUNK)])
        plsc.subcore_barrier()

        def prefetch(c, ipq_v, igh_v, dst_v, bufpq, bufgh, sem0, sem1, sem2):
            off = sid * ew + c * CHUNK
            ci1 = pltpu.async_copy(ipq_hbm.at[pl.ds(off, CHUNK)], ipq_v, sem0)
            ci2 = pltpu.async_copy(igh_hbm.at[pl.ds(off, CHUNK)], igh_v, sem0)
            ci3 = pltpu.async_copy(dst_hbm.at[pl.ds(off, CHUNK)], dst_v, sem0)
            ci1.wait()
            ci2.wait()
            ci3.wait()
            cp1 = pltpu.async_copy(pq_hbm.at[ipq_v], bufpq, sem1)
            cp2 = pltpu.async_copy(gh_hbm.at[igh_v], bufgh, sem2)
            for j in range(CHUNK // L):
                d = dst_v[pl.ds(j * L, L)] - dst_lo
                oob = (d < 0) | (d >= NHALF)
                dst_v[pl.ds(j * L, L)] = jnp.where(oob, NHALF, d)
            return cp1, cp2

        def commit(dst_v, bufpq, bufgh, cp1, cp2, sem0):
            cp1.wait()
            cs1 = pltpu.async_copy(bufpq, acc_sh.at[dst_v], sem0, add=True)
            cp2.wait()
            cs2 = pltpu.async_copy(bufgh, acc_sh.at[dst_v], sem0, add=True)
            cs1.wait()
            cs2.wait()

        cpA = prefetch(0, ipqA, ighA, dstA, bufpqA, bufghA, semA0, semA1, semA2)

        def body(i, carry):
            cB = 2 * i + 1
            cpB = prefetch(cB, ipqB, ighB, dstB, bufpqB, bufghB, semB0, semB1, semB2)
            commit(dstA, bufpqA, bufghA,
                   pltpu.make_async_copy(pq_hbm.at[ipqA], bufpqA, semA1),
                   pltpu.make_async_copy(gh_hbm.at[ighA], bufghA, semA2),
                   semA0)
            cpA2 = prefetch(cB + 1, ipqA, ighA, dstA, bufpqA, bufghA,
                            semA0, semA1, semA2)
            commit(dstB, bufpqB, bufghB,
                   pltpu.make_async_copy(pq_hbm.at[ipqB], bufpqB, semB1),
                   pltpu.make_async_copy(gh_hbm.at[ighB], bufghB, semB2),
                   semB0)
            return carry

        lax.fori_loop(0, n_chunks // 2 - 1, body, 0)
        cpB = prefetch(n_chunks - 1, ipqB, ighB, dstB, bufpqB, bufghB,
                       semB0, semB1, semB2)
        commit(dstA, bufpqA, bufghA,
               pltpu.make_async_copy(pq_hbm.at[ipqA], bufpqA, semA1),
               pltpu.make_async_copy(gh_hbm.at[ighA], bufghA, semA2),
               semA0)
        commit(dstB, bufpqB, bufghB, cpB[0], cpB[1], semB0)
        plsc.subcore_barrier()

        row0 = sid * FSTRIPE
        out0 = cid * NHALF + row0
        for k in range(FSTRIPE // CHUNK):
            pltpu.sync_copy(acc_sh.at[pl.ds(row0 + k * CHUNK, CHUNK)], bufpqA)
            pltpu.sync_copy(bufpqA, acc_out.at[pl.ds(out0 + k * CHUNK, CHUNK)])

    return sck


def kernel(ent_emb, rel_emb, edge_index, edge_rel, edge_inv,
           W_I, b_I, W_O, b_O, W_S, b_S, W_R, b_R):
    n, ent_d = ent_emb.shape
    r_cnt, rel_d = rel_emb.shape
    e_total = edge_rel.shape[0]
    assert (n, ent_d, rel_d) == (N, ENT, REL)
    assert e_total % (NS * CHUNK) == 0 and e_total % EB == 0

    src = edge_index[0]
    dst = edge_index[1]

    ent_pad = jnp.zeros((NPAD, ENT), jnp.float32).at[:n].set(ent_emb)
    gh, s_out = pl.pallas_call(
        _prep_ent_body,
        grid=(NPAD // BA,),
        in_specs=[
            pl.BlockSpec((BA, ENT), lambda i: (i, 0)),
            pl.BlockSpec((ENT, ENT), lambda i: (0, 0)),
            pl.BlockSpec((ENT, ENT), lambda i: (0, 0)),
            pl.BlockSpec((ENT, ENT), lambda i: (0, 0)),
            pl.BlockSpec((1, ENT), lambda i: (0, 0)),
        ],
        out_specs=[
            pl.BlockSpec((2, BA, ENT), lambda i: (0, i, 0)),
            pl.BlockSpec((BA, ENT), lambda i: (i, 0)),
        ],
        out_shape=[
            jax.ShapeDtypeStruct((2, NPAD, ENT), jnp.float32),
            jax.ShapeDtypeStruct((NPAD, ENT), jnp.float32),
        ],
    )(ent_pad, W_I[:, REL:].T, W_O[:, REL:].T, W_S.T, b_S[None, :])

    rel_pad = jnp.zeros((RPAD, rel_d), jnp.float32).at[:r_cnt].set(rel_emb)
    pq, rel_new_pad = pl.pallas_call(
        _prep_rel_body,
        out_shape=[
            jax.ShapeDtypeStruct((2, RPAD, ENT), jnp.float32),
            jax.ShapeDtypeStruct((RPAD, REL), jnp.float32),
        ],
    )(rel_pad, W_I[:, :REL].T, W_O[:, :REL].T, W_R.T,
      b_I[None, :], b_O[None, :], b_R[None, :])

    deg2d = pl.pallas_call(
        _deg_hist_body,
        grid=(e_total // EB,),
        in_specs=[pl.BlockSpec((1, EB, 1), lambda k: (k, 0, 0))],
        out_specs=pl.BlockSpec((DH, DL), lambda k: (0, 0)),
        out_shape=jax.ShapeDtypeStruct((DH, DL), jnp.float32),
    )(dst.reshape(e_total // EB, EB, 1))

    idx_pq = edge_rel + edge_inv * RPAD
    idx_gh = src + edge_inv * NPAD
    zrow = jnp.zeros((CHUNK, ENT), jnp.float32)
    acc = _edge_sc_kernel(e_total)(
        pq.reshape(2 * RPAD, ENT), gh.reshape(2 * NPAD, ENT),
        idx_pq, idx_gh, dst, zrow)

    ent_new_pad = pl.pallas_call(
        _post_body,
        grid=(NPAD // BA,),
        in_specs=[
            pl.BlockSpec((BA, ENT), lambda i: (i, 0)),
            pl.BlockSpec((BA, 1), lambda i: (i, 0)),
            pl.BlockSpec((BA, ENT), lambda i: (i, 0)),
        ],
        out_specs=pl.BlockSpec((BA, ENT), lambda i: (i, 0)),
        out_shape=jax.ShapeDtypeStruct((NPAD, ENT), jnp.float32),
    )(acc, deg2d.reshape(NPAD, 1), s_out)

    return ent_new_pad[:n], rel_new_pad[:r_cnt]
